# baseline (speedup 1.0000x reference)
import functools

import jax
import jax.numpy as jnp
from jax import lax
from jax.experimental import pallas as pl
from jax.experimental.pallas import tpu as pltpu
from jax.experimental.pallas import tpu_sc as plsc

_NC = 2
_NS = 16
_NW = _NC * _NS


@functools.lru_cache(maxsize=None)
def _make_wtrans(V, D):
    mesh = plsc.VectorSubcoreMesh(core_axis_name="c", subcore_axis_name="s")
    rpw = (-(-V // _NW) + 7) // 8 * 8
    rlast = V - (_NW - 1) * rpw
    CH = 632
    n_ch = -(-rpw // CH)

    @functools.partial(
        pl.kernel,
        mesh=mesh,
        out_type=jax.ShapeDtypeStruct((V * D,), jnp.float32),
        scratch_types=[
            pltpu.VMEM((2, D, CH), jnp.float32),
            pltpu.VMEM((2, CH * D), jnp.float32),
            pltpu.SemaphoreType.DMA((2,)),
            pltpu.SemaphoreType.DMA((2,)),
        ],
        compiler_params=pltpu.CompilerParams(
            use_tc_tiling_on_sc=False, needs_layout_passes=False
        ),
    )
    def wtrans_k(wT_hbm, wflat_hbm, in_v, out_v, sem_i, sem_o):
        wid = lax.axis_index("s") * _NC + lax.axis_index("c")
        r0 = wid * rpw
        rw = jnp.where(wid == _NW - 1, rlast, rpw)

        def cstart(c):
            return r0 + jnp.minimum(c * CH, rw - CH)

        def fire_in(c, b):
            pltpu.async_copy(
                wT_hbm.at[:, pl.ds(cstart(c), CH)], in_v.at[b], sem_i.at[b]
            )

        def wait_in(c, b):
            pltpu.make_async_copy(
                wT_hbm.at[:, pl.ds(cstart(c), CH)], in_v.at[b], sem_i.at[b]
            ).wait()

        def fire_out(c, b):
            pltpu.async_copy(
                out_v.at[b],
                wflat_hbm.at[pl.ds(cstart(c) * D, CH * D)],
                sem_o.at[b],
            )

        def wait_out(c, b):
            pltpu.make_async_copy(
                out_v.at[b],
                wflat_hbm.at[pl.ds(cstart(c) * D, CH * D)],
                sem_o.at[b],
            ).wait()

        d_lo = lax.iota(jnp.int32, 16)
        d_hi = d_lo + 16

        def transpose(b):
            slab = in_v.at[b]

            def row_body(k, carry):
                for q in range(4):
                    r = 4 * k + q
                    rvec = jnp.zeros((16,), jnp.int32) + r
                    v0 = plsc.load_gather(slab, [d_lo, rvec])
                    v1 = plsc.load_gather(slab, [d_hi, rvec])
                    out_v[b, pl.ds(r * D, 16)] = v0
                    out_v[b, pl.ds(r * D + 16, 16)] = v1
                return carry

            lax.fori_loop(0, CH // 4, row_body, 0)

        fire_in(jnp.int32(0), 0)

        def body(u, carry):
            c0 = 2 * u
            c1 = 2 * u + 1
            fire_in(c1, 1)
            wait_in(c0, 0)

            @pl.when(u >= 1)
            def _():
                wait_out(c0 - 2, 0)

            transpose(0)
            fire_out(c0, 0)

            @pl.when(c0 + 2 < n_ch)
            def _():
                fire_in(c0 + 2, 0)

            wait_in(c1, 1)

            @pl.when(u >= 1)
            def _():
                wait_out(c1 - 2, 1)

            transpose(1)
            fire_out(c1, 1)
            return carry

        lax.fori_loop(0, n_ch // 2, body, 0)

        wait_out(jnp.int32(n_ch - 2), 0)
        wait_out(jnp.int32(n_ch - 1), 1)

    return wtrans_k


@functools.lru_cache(maxsize=None)
def _make_lookup(B, H, D):
    mesh = plsc.VectorSubcoreMesh(core_axis_name="c", subcore_axis_name="s")
    bw = B // _NW

    @functools.partial(
        pl.kernel,
        mesh=mesh,
        out_type=jax.ShapeDtypeStruct((H, D, B), jnp.float32),
        scratch_types=[
            pltpu.VMEM((H, bw), jnp.int32),
            pltpu.VMEM((2, bw, D), jnp.float32),
            pltpu.VMEM((2, D, bw + 1), jnp.float32),
            pltpu.SemaphoreType.DMA((2,)),
            pltpu.SemaphoreType.DMA((2,)),
        ],
        compiler_params=pltpu.CompilerParams(
            use_tc_tiling_on_sc=False, needs_layout_passes=False
        ),
    )
    def lookup_k(idxT_hbm, w_hbm, outT_hbm, idx_v, blk_v, out_v, sem_g, sem_o):
        wid = lax.axis_index("s") * _NC + lax.axis_index("c")
        b0 = wid * bw
        pltpu.sync_copy(idxT_hbm.at[:, pl.ds(b0, bw)], idx_v)

        def fire_gather(l, gb):
            pltpu.async_copy(w_hbm.at[idx_v.at[l]], blk_v.at[gb], sem_g.at[gb])

        def wait_gather(l, gb):
            pltpu.make_async_copy(
                w_hbm.at[idx_v.at[l]], blk_v.at[gb], sem_g.at[gb]
            ).wait()

        def fire_out(l, ob):
            pltpu.async_copy(
                out_v.at[ob, :, pl.ds(0, bw)],
                outT_hbm.at[l, :, pl.ds(b0, bw)],
                sem_o.at[ob],
            )

        def wait_out(l, ob):
            pltpu.make_async_copy(
                out_v.at[ob, :, pl.ds(0, bw)],
                outT_hbm.at[l, :, pl.ds(b0, bw)],
                sem_o.at[ob],
            ).wait()

        d_lo = lax.iota(jnp.int32, 16)
        d_hi = d_lo + 16

        def transpose(b):
            rows = blk_v.at[b]
            outp = out_v.at[b]

            def row_body(k, carry):
                for q in range(4):
                    bp = 4 * k + q
                    bvec = jnp.zeros((16,), jnp.int32) + bp
                    v0 = rows[bp, pl.ds(0, 16)]
                    v1 = rows[bp, pl.ds(16, 16)]
                    plsc.store_scatter(outp, [d_lo, bvec], v0)
                    plsc.store_scatter(outp, [d_hi, bvec], v1)
                return carry

            lax.fori_loop(0, bw // 4, row_body, 0)

        fire_gather(jnp.int32(0), 0)

        def body(u, carry):
            la = 2 * u
            lb = 2 * u + 1
            fire_gather(lb, 1)
            wait_gather(la, 0)

            @pl.when(u >= 1)
            def _():
                wait_out(la - 2, 0)

            transpose(0)
            fire_out(la, 0)

            @pl.when(lb + 1 < H)
            def _():
                fire_gather(lb + 1, 0)

            wait_gather(lb, 1)

            @pl.when(u >= 1)
            def _():
                wait_out(lb - 2, 1)

            transpose(1)
            fire_out(lb, 1)
            return carry

        lax.fori_loop(0, H // 2, body, 0)

        wait_out(jnp.int32(H - 2), 0)
        wait_out(jnp.int32(H - 1), 1)

    return lookup_k


def kernel(input, weight):
    B, H = input.shape
    V, D = weight.shape
    idxT = input.T.astype(jnp.int32)
    wflat = _make_wtrans(V, D)(weight.T)
    outT = _make_lookup(B, H, D)(idxT, wflat.reshape(V, D))
    return outT.transpose(2, 0, 1)


# speedup vs baseline: 4.0774x; 4.0774x over previous
import functools

import jax
import jax.numpy as jnp
from jax import lax
from jax.experimental import pallas as pl
from jax.experimental.pallas import tpu as pltpu
from jax.experimental.pallas import tpu_sc as plsc

_NC = 2
_NS = 16
_NW = _NC * _NS


@functools.lru_cache(maxsize=None)
def _make_lookup(B, H, D):
    mesh = plsc.VectorSubcoreMesh(core_axis_name="c", subcore_axis_name="s")
    bw = B // _NW

    @functools.partial(
        pl.kernel,
        mesh=mesh,
        out_type=jax.ShapeDtypeStruct((H, D, B), jnp.float32),
        scratch_types=[
            pltpu.VMEM((H, bw), jnp.int32),
            pltpu.VMEM((2, bw, D), jnp.float32),
            pltpu.VMEM((2, D, bw + 1), jnp.float32),
            pltpu.SemaphoreType.DMA((2,)),
            pltpu.SemaphoreType.DMA((2,)),
        ],
        compiler_params=pltpu.CompilerParams(
            use_tc_tiling_on_sc=False, needs_layout_passes=False
        ),
    )
    def lookup_k(idxT_hbm, w_hbm, outT_hbm, idx_v, blk_v, out_v, sem_g, sem_o):
        wid = lax.axis_index("s") * _NC + lax.axis_index("c")
        b0 = wid * bw
        pltpu.sync_copy(idxT_hbm.at[:, pl.ds(b0, bw)], idx_v)

        def fire_gather(l, gb):
            pltpu.async_copy(w_hbm.at[idx_v.at[l]], blk_v.at[gb], sem_g.at[gb])

        def wait_gather(l, gb):
            pltpu.make_async_copy(
                w_hbm.at[idx_v.at[l]], blk_v.at[gb], sem_g.at[gb]
            ).wait()

        def fire_out(l, ob):
            pltpu.async_copy(
                out_v.at[ob, :, pl.ds(0, bw)],
                outT_hbm.at[l, :, pl.ds(b0, bw)],
                sem_o.at[ob],
            )

        def wait_out(l, ob):
            pltpu.make_async_copy(
                out_v.at[ob, :, pl.ds(0, bw)],
                outT_hbm.at[l, :, pl.ds(b0, bw)],
                sem_o.at[ob],
            ).wait()

        d_lo = lax.iota(jnp.int32, 16)
        d_hi = d_lo + 16

        def transpose(b):
            rows = blk_v.at[b]
            outp = out_v.at[b]

            def row_body(k, carry):
                for q in range(4):
                    bp = 4 * k + q
                    bvec = jnp.zeros((16,), jnp.int32) + bp
                    v0 = rows[bp, pl.ds(0, 16)]
                    v1 = rows[bp, pl.ds(16, 16)]
                    plsc.store_scatter(outp, [d_lo, bvec], v0)
                    plsc.store_scatter(outp, [d_hi, bvec], v1)
                return carry

            lax.fori_loop(0, bw // 4, row_body, 0)

        fire_gather(jnp.int32(0), 0)

        def body(u, carry):
            la = 2 * u
            lb = 2 * u + 1
            fire_gather(lb, 1)
            wait_gather(la, 0)

            @pl.when(u >= 1)
            def _():
                wait_out(la - 2, 0)

            transpose(0)
            fire_out(la, 0)

            @pl.when(lb + 1 < H)
            def _():
                fire_gather(lb + 1, 0)

            wait_gather(lb, 1)

            @pl.when(u >= 1)
            def _():
                wait_out(lb - 2, 1)

            transpose(1)
            fire_out(lb, 1)
            return carry

        lax.fori_loop(0, H // 2, body, 0)

        wait_out(jnp.int32(H - 2), 0)
        wait_out(jnp.int32(H - 1), 1)

    return lookup_k


def kernel(input, weight):
    B, H = input.shape
    V, D = weight.shape
    idxT = input.T.astype(jnp.int32)
    outT = _make_lookup(B, H, D)(idxT, weight)
    return outT.transpose(2, 0, 1)


# speedup vs baseline: 4.6510x; 1.1407x over previous
import functools

import jax
import jax.numpy as jnp
from jax import lax
from jax.experimental import pallas as pl
from jax.experimental.pallas import tpu as pltpu
from jax.experimental.pallas import tpu_sc as plsc

_NC = 2
_NS = 16
_NW = _NC * _NS


@functools.lru_cache(maxsize=None)
def _make_lookup(B, H, D):
    mesh = plsc.VectorSubcoreMesh(core_axis_name="c", subcore_axis_name="s")
    bw = B // _NW

    @functools.partial(
        pl.kernel,
        mesh=mesh,
        out_type=jax.ShapeDtypeStruct((H, D // 8, B // 128, 8, 128),
                                      jnp.float32),
        scratch_types=[
            pltpu.VMEM((H, bw), jnp.int32),
            pltpu.VMEM((2, bw, D), jnp.float32),
            pltpu.VMEM((2, D // 8, bw // 128, 8, 136), jnp.float32),
            pltpu.SemaphoreType.DMA((2,)),
            pltpu.SemaphoreType.DMA((2,)),
        ],
        compiler_params=pltpu.CompilerParams(
            use_tc_tiling_on_sc=False, needs_layout_passes=False
        ),
    )
    def lookup_k(idxT_hbm, w_hbm, outT_hbm, idx_v, blk_v, out_v, sem_g, sem_o):
        wid = lax.axis_index("s") * _NC + lax.axis_index("c")
        b0 = wid * bw
        pltpu.sync_copy(idxT_hbm.at[:, pl.ds(b0, bw)], idx_v)

        def fire_gather(l, gb):
            pltpu.async_copy(w_hbm.at[idx_v.at[l]], blk_v.at[gb], sem_g.at[gb])

        def wait_gather(l, gb):
            pltpu.make_async_copy(
                w_hbm.at[idx_v.at[l]], blk_v.at[gb], sem_g.at[gb]
            ).wait()

        tc0 = b0 // 128

        def fire_out(l, ob):
            pltpu.async_copy(
                out_v.at[ob, :, :, :, pl.ds(0, 128)],
                outT_hbm.at[l, :, pl.ds(tc0, bw // 128)],
                sem_o.at[ob],
            )

        def wait_out(l, ob):
            pltpu.make_async_copy(
                out_v.at[ob, :, :, :, pl.ds(0, 128)],
                outT_hbm.at[l, :, pl.ds(tc0, bw // 128)],
                sem_o.at[ob],
            ).wait()

        d_lo = lax.iota(jnp.int32, 16)
        d_hi = d_lo + 16
        tr_lo, dr_lo = d_lo // 8, d_lo % 8
        tr_hi, dr_hi = d_hi // 8, d_hi % 8

        def transpose(b):
            rows = blk_v.at[b]
            outp = out_v.at[b]

            def row_body(k, carry):
                for q in range(4):
                    bp = 4 * k + q
                    tcv = jnp.zeros((16,), jnp.int32) + bp // 128
                    bcv = jnp.zeros((16,), jnp.int32) + bp % 128
                    v0 = rows[bp, pl.ds(0, 16)]
                    v1 = rows[bp, pl.ds(16, 16)]
                    plsc.store_scatter(outp, [tr_lo, tcv, dr_lo, bcv], v0)
                    plsc.store_scatter(outp, [tr_hi, tcv, dr_hi, bcv], v1)
                return carry

            lax.fori_loop(0, bw // 4, row_body, 0)

        fire_gather(jnp.int32(0), 0)

        def body(u, carry):
            la = 2 * u
            lb = 2 * u + 1
            fire_gather(lb, 1)
            wait_gather(la, 0)

            @pl.when(u >= 1)
            def _():
                wait_out(la - 2, 0)

            transpose(0)
            fire_out(la, 0)

            @pl.when(lb + 1 < H)
            def _():
                fire_gather(lb + 1, 0)

            wait_gather(lb, 1)

            @pl.when(u >= 1)
            def _():
                wait_out(lb - 2, 1)

            transpose(1)
            fire_out(lb, 1)
            return carry

        lax.fori_loop(0, H // 2, body, 0)

        wait_out(jnp.int32(H - 2), 0)
        wait_out(jnp.int32(H - 1), 1)

    return lookup_k


def kernel(input, weight):
    B, H = input.shape
    V, D = weight.shape
    idxT = input.T.astype(jnp.int32)
    out5 = _make_lookup(B, H, D)(idxT, weight)
    return out5.transpose(2, 4, 0, 1, 3).reshape(B, H, D)


# speedup vs baseline: 4.6551x; 1.0009x over previous
import functools

import jax
import jax.numpy as jnp
from jax import lax
from jax.experimental import pallas as pl
from jax.experimental.pallas import tpu as pltpu
from jax.experimental.pallas import tpu_sc as plsc

_NC = 2
_NS = 16
_NW = _NC * _NS


@functools.lru_cache(maxsize=None)
def _make_lookup(B, H, D):
    mesh = plsc.VectorSubcoreMesh(core_axis_name="c", subcore_axis_name="s")
    bw = B // _NW

    @functools.partial(
        pl.kernel,
        mesh=mesh,
        out_type=jax.ShapeDtypeStruct((H, D // 8, B // 128, 8, 128),
                                      jnp.float32),
        scratch_types=[
            pltpu.VMEM((H, bw), jnp.int32),
            pltpu.VMEM((2, bw, D), jnp.float32),
            pltpu.VMEM((2, D // 8, bw // 128 + 1, 8, 136), jnp.float32),
            pltpu.SemaphoreType.DMA((2,)),
            pltpu.SemaphoreType.DMA((2,)),
        ],
        compiler_params=pltpu.CompilerParams(
            use_tc_tiling_on_sc=False, needs_layout_passes=False
        ),
    )
    def lookup_k(idxT_hbm, w_hbm, outT_hbm, idx_v, blk_v, out_v, sem_g, sem_o):
        wid = lax.axis_index("s") * _NC + lax.axis_index("c")
        b0 = wid * bw
        pltpu.sync_copy(idxT_hbm.at[:, pl.ds(b0, bw)], idx_v)

        def fire_gather(l, gb):
            pltpu.async_copy(w_hbm.at[idx_v.at[l]], blk_v.at[gb], sem_g.at[gb])

        def wait_gather(l, gb):
            pltpu.make_async_copy(
                w_hbm.at[idx_v.at[l]], blk_v.at[gb], sem_g.at[gb]
            ).wait()

        tc0 = b0 // 128

        def fire_out(l, ob):
            pltpu.async_copy(
                out_v.at[ob, :, pl.ds(0, bw // 128), :, pl.ds(0, 128)],
                outT_hbm.at[l, :, pl.ds(tc0, bw // 128)],
                sem_o.at[ob],
            )

        def wait_out(l, ob):
            pltpu.make_async_copy(
                out_v.at[ob, :, pl.ds(0, bw // 128), :, pl.ds(0, 128)],
                outT_hbm.at[l, :, pl.ds(tc0, bw // 128)],
                sem_o.at[ob],
            ).wait()

        d_lo = lax.iota(jnp.int32, 16)
        d_hi = d_lo + 16
        tr_lo, dr_lo = d_lo // 8, d_lo % 8
        tr_hi, dr_hi = d_hi // 8, d_hi % 8

        def transpose(b):
            rows = blk_v.at[b]
            outp = out_v.at[b]

            def row_body(k, carry):
                for q in range(4):
                    bp = 4 * k + q
                    tcv = jnp.zeros((16,), jnp.int32) + bp // 128
                    bcv = jnp.zeros((16,), jnp.int32) + bp % 128
                    v0 = rows[bp, pl.ds(0, 16)]
                    v1 = rows[bp, pl.ds(16, 16)]
                    plsc.store_scatter(outp, [tr_lo, tcv, dr_lo, bcv], v0)
                    plsc.store_scatter(outp, [tr_hi, tcv, dr_hi, bcv], v1)
                return carry

            lax.fori_loop(0, bw // 4, row_body, 0)

        fire_gather(jnp.int32(0), 0)

        def body(u, carry):
            la = 2 * u
            lb = 2 * u + 1
            fire_gather(lb, 1)
            wait_gather(la, 0)

            @pl.when(u >= 1)
            def _():
                wait_out(la - 2, 0)

            transpose(0)
            fire_out(la, 0)

            @pl.when(lb + 1 < H)
            def _():
                fire_gather(lb + 1, 0)

            wait_gather(lb, 1)

            @pl.when(u >= 1)
            def _():
                wait_out(lb - 2, 1)

            transpose(1)
            fire_out(lb, 1)
            return carry

        lax.fori_loop(0, H // 2, body, 0)

        wait_out(jnp.int32(H - 2), 0)
        wait_out(jnp.int32(H - 1), 1)

    return lookup_k


def kernel(input, weight):
    B, H = input.shape
    V, D = weight.shape
    idxT = input.T.astype(jnp.int32)
    out5 = _make_lookup(B, H, D)(idxT, weight)
    return out5.transpose(2, 4, 0, 1, 3).reshape(B, H, D)
